# Initial kernel scaffold; baseline (speedup 1.0000x reference)
import jax
import jax.numpy as jnp
from jax.experimental import pallas as pl


def kernel(word_id_sequence, word_table, fsa_table, fsa_transitions):
    raise NotImplementedError("write your pallas kernel here")


# speedup vs baseline: 4.7173x; 4.7173x over previous
import functools
import math

import jax
import jax.numpy as jnp
from jax import lax
from jax.experimental import pallas as pl
from jax.experimental.pallas import tpu as pltpu, tpu_sc as plsc
import numpy as np

VOCAB = 100000
WORD_DIM = 128
FSA_DIM = 64
NUM_STATES = 256
B = 1024
S = 200
OUT_DIM = WORD_DIM + FSA_DIM
SCALE = math.sqrt(float(WORD_DIM))
SPAD = 208


def _pe_table(seq_len, dim):
    pos = jnp.arange(seq_len, dtype=jnp.float32)[:, None]
    div = jnp.exp(jnp.arange(0, dim, 2, dtype=jnp.float32) * (-np.log(10000.0) / dim))
    ang = pos * div[None, :]
    pe = jnp.zeros((seq_len, dim), dtype=jnp.float32)
    pe = pe.at[:, 0::2].set(jnp.sin(ang))
    pe = pe.at[:, 1::2].set(jnp.cos(ang))
    return pe


def _make_sc_kernel():
    mesh = plsc.VectorSubcoreMesh(core_axis_name="c", subcore_axis_name="s")
    info = plsc.get_sparse_core_info()
    NC, NS = info.num_cores, info.num_subcores
    NW = NC * NS
    EPW = B // NW

    @functools.partial(
        pl.kernel,
        out_type=jax.ShapeDtypeStruct((B, S, OUT_DIM), jnp.float32),
        mesh=mesh,
        compiler_params=pltpu.CompilerParams(
            use_tc_tiling_on_sc=False, needs_layout_passes=False),
        scratch_types=[
            pltpu.VMEM((EPW, S), jnp.int32),
            pltpu.VMEM((SPAD, EPW), jnp.int32),
            pltpu.VMEM((SPAD,), jnp.int32),
            pltpu.VMEM((EPW,), jnp.int32),
            pltpu.VMEM((S, WORD_DIM), jnp.float32),
            pltpu.VMEM((SPAD, FSA_DIM), jnp.float32),
            pltpu.VMEM((S, WORD_DIM), jnp.float32),
            pltpu.SemaphoreType.DMA,
            pltpu.SemaphoreType.DMA,
            pltpu.SemaphoreType.DMA,
        ],
    )
    def sc_kernel(wids, wtab, ftab, trans, pe, out,
                  syms, hist, fidx, idxbuf, wbuf, fbuf, pe_v,
                  sem_a, sem_w, sem_f):
        wid = lax.axis_index("s") * NC + lax.axis_index("c")
        base = wid * EPW
        lanes = lax.iota(jnp.int32, 16)

        pltpu.sync_copy(wids.at[pl.ds(base, EPW)], syms)
        pltpu.sync_copy(pe, pe_v)

        def scan_step(t, _):
            for g in range(EPW // 16):
                e_idx = lanes + (16 * g)
                t_idx = jnp.full((16,), t, dtype=jnp.int32)
                sym = plsc.load_gather(syms, [e_idx, t_idx])
                prev = hist[lax.max(t - 1, 0), pl.ds(16 * g, 16)]
                state = jnp.where(t == 0, jnp.int32(0), prev)
                idxbuf[pl.ds(16 * g, 16)] = state * jnp.int32(VOCAB) + sym
            pltpu.async_copy(trans.at[idxbuf], hist.at[t], sem_a).wait()
            return 0

        lax.fori_loop(0, S, scan_step, 0, unroll=False)

        def emit_example(e, _):
            for j in range(SPAD // 16):
                t_idx = lanes + (16 * j)
                e_idx = jnp.full((16,), e, dtype=jnp.int32)
                v = plsc.load_gather(hist, [t_idx, e_idx])
                v = jnp.where(t_idx < S, v, jnp.int32(0))
                fidx[pl.ds(16 * j, 16)] = v
            cw0 = pltpu.async_copy(
                wtab.at[syms.at[e, pl.ds(0, 128)]], wbuf.at[pl.ds(0, 128)], sem_w)
            cw1 = pltpu.async_copy(
                wtab.at[syms.at[e, pl.ds(128, S - 128)]],
                wbuf.at[pl.ds(128, S - 128)], sem_w)
            cf0 = pltpu.async_copy(
                ftab.at[fidx.at[pl.ds(0, 128)]], fbuf.at[pl.ds(0, 128)], sem_f)
            cf1 = pltpu.async_copy(
                ftab.at[fidx.at[pl.ds(128, SPAD - 128)]],
                fbuf.at[pl.ds(128, SPAD - 128)], sem_f)
            cw0.wait(); cw1.wait(); cf0.wait(); cf1.wait()

            def fuse_row(t, _):
                for j in range(WORD_DIM // 16):
                    w = wbuf[t, pl.ds(16 * j, 16)]
                    p = pe_v[t, pl.ds(16 * j, 16)]
                    wbuf[t, pl.ds(16 * j, 16)] = w * jnp.float32(SCALE) + p
                return 0

            lax.fori_loop(0, S, fuse_row, 0, unroll=False)

            pltpu.sync_copy(wbuf, out.at[base + e, :, pl.ds(0, WORD_DIM)])
            pltpu.sync_copy(fbuf.at[pl.ds(0, S)],
                            out.at[base + e, :, pl.ds(WORD_DIM, FSA_DIM)])
            return 0

        lax.fori_loop(0, EPW, emit_example, 0, unroll=False)

    return sc_kernel


def kernel(word_id_sequence, word_table, fsa_table, fsa_transitions):
    pe = _pe_table(S, WORD_DIM)
    trans_flat = fsa_transitions.reshape(-1)
    sc = _make_sc_kernel()
    return sc(word_id_sequence, word_table, fsa_table, trans_flat, pe)


# speedup vs baseline: 6.7721x; 1.4356x over previous
import functools
import math

import jax
import jax.numpy as jnp
from jax import lax
from jax.experimental import pallas as pl
from jax.experimental.pallas import tpu as pltpu, tpu_sc as plsc
import numpy as np

VOCAB = 100000
WORD_DIM = 128
FSA_DIM = 64
NUM_STATES = 256
B = 1024
S = 200
OUT_DIM = WORD_DIM + FSA_DIM
SCALE = math.sqrt(float(WORD_DIM))
SPAD = 208

SBLK = 8
BBLK = 512


def _pe_table(seq_len, dim):
    pos = jnp.arange(seq_len, dtype=jnp.float32)[:, None]
    div = jnp.exp(jnp.arange(0, dim, 2, dtype=jnp.float32) * (-np.log(10000.0) / dim))
    ang = pos * div[None, :]
    pe = jnp.zeros((seq_len, dim), dtype=jnp.float32)
    pe = pe.at[:, 0::2].set(jnp.sin(ang))
    pe = pe.at[:, 1::2].set(jnp.cos(ang))
    return pe


def _make_sc_kernel():
    mesh = plsc.VectorSubcoreMesh(core_axis_name="c", subcore_axis_name="s")
    info = plsc.get_sparse_core_info()
    NC, NS = info.num_cores, info.num_subcores
    NW = NC * NS
    EPW = B // NW

    @functools.partial(
        pl.kernel,
        out_type=(
            jax.ShapeDtypeStruct((B, S, WORD_DIM), jnp.float32),
            jax.ShapeDtypeStruct((S, B), jnp.int32),
        ),
        mesh=mesh,
        compiler_params=pltpu.CompilerParams(
            use_tc_tiling_on_sc=False, needs_layout_passes=False),
        scratch_types=[
            pltpu.VMEM((S, EPW), jnp.int32),
            pltpu.VMEM((SPAD, EPW), jnp.int32),
            pltpu.VMEM((SPAD,), jnp.int32),
            pltpu.VMEM((EPW,), jnp.int32),
            pltpu.VMEM((S, WORD_DIM), jnp.float32),
            pltpu.SemaphoreType.DMA,
            pltpu.SemaphoreType.DMA,
        ],
    )
    def sc_kernel(wids_t, wtab, trans, word_raw, states_t,
                  syms, hist, syms_e, idxbuf, wbuf, sem_a, sem_w):
        wid = lax.axis_index("s") * NC + lax.axis_index("c")
        base = wid * EPW
        lanes = lax.iota(jnp.int32, 16)

        pltpu.sync_copy(wids_t.at[:, pl.ds(base, EPW)], syms)

        def scan_step(t, _):
            for g in range(EPW // 16):
                sym = syms[t, pl.ds(16 * g, 16)]
                prev = hist[lax.max(t - 1, 0), pl.ds(16 * g, 16)]
                state = jnp.where(t == 0, jnp.int32(0), prev)
                idxbuf[pl.ds(16 * g, 16)] = state * jnp.int32(VOCAB) + sym
            pltpu.async_copy(trans.at[idxbuf], hist.at[t], sem_a).wait()
            return 0

        lax.fori_loop(0, S, scan_step, 0, unroll=False)

        pltpu.sync_copy(hist.at[pl.ds(0, S)], states_t.at[:, pl.ds(base, EPW)])

        def emit_example(e, _):
            for j in range(SPAD // 16):
                t_idx = lanes + (16 * j)
                e_idx = jnp.full((16,), e, dtype=jnp.int32)
                v = plsc.load_gather(syms, [jnp.minimum(t_idx, S - 1), e_idx])
                syms_e[pl.ds(16 * j, 16)] = v
            cw0 = pltpu.async_copy(
                wtab.at[syms_e.at[pl.ds(0, 128)]], wbuf.at[pl.ds(0, 128)], sem_w)
            cw1 = pltpu.async_copy(
                wtab.at[syms_e.at[pl.ds(128, S - 128)]],
                wbuf.at[pl.ds(128, S - 128)], sem_w)
            cw0.wait(); cw1.wait()
            pltpu.sync_copy(wbuf, word_raw.at[base + e])
            return 0

        lax.fori_loop(0, EPW, emit_example, 0, unroll=False)

    return sc_kernel


def _tc_assemble(word_raw, states_t, fsa_table, pe):
    grid = (S // SBLK, B // BBLK)

    def body(w_ref, st_ref, ft_ref, pe_ref, o_ref):
        ident = jnp.eye(WORD_DIM, dtype=jnp.float32)
        qiota = lax.broadcasted_iota(jnp.int32, (NUM_STATES, BBLK), 0)
        for i in range(SBLK):
            w = w_ref[:, i, :]
            wt = lax.dot_general(ident, w, (((1,), (1,)), ((), ())),
                                 precision=lax.Precision.HIGHEST,
                                 preferred_element_type=jnp.float32)
            o_ref[i, 0:WORD_DIM, :] = wt * jnp.float32(SCALE) + pe_ref[i][:, None]
            onehot = (qiota == st_ref[i][None, :]).astype(jnp.float32)
            fsa = lax.dot_general(ft_ref[...], onehot, (((0,), (0,)), ((), ())),
                                  precision=lax.Precision.HIGHEST,
                                  preferred_element_type=jnp.float32)
            o_ref[i, WORD_DIM:OUT_DIM, :] = fsa

    return pl.pallas_call(
        body,
        grid=grid,
        in_specs=[
            pl.BlockSpec((BBLK, SBLK, WORD_DIM), lambda s, b: (b, s, 0)),
            pl.BlockSpec((SBLK, BBLK), lambda s, b: (s, b)),
            pl.BlockSpec((NUM_STATES, FSA_DIM), lambda s, b: (0, 0)),
            pl.BlockSpec((SBLK, WORD_DIM), lambda s, b: (s, 0)),
        ],
        out_specs=pl.BlockSpec((SBLK, OUT_DIM, BBLK), lambda s, b: (s, 0, b)),
        out_shape=jax.ShapeDtypeStruct((S, OUT_DIM, B), jnp.float32),
        compiler_params=pltpu.CompilerParams(
            dimension_semantics=("arbitrary", "arbitrary")),
    )(word_raw, states_t, fsa_table, pe)


def kernel(word_id_sequence, word_table, fsa_table, fsa_transitions):
    pe = _pe_table(S, WORD_DIM)
    wids_t = word_id_sequence.T
    trans_flat = fsa_transitions.reshape(-1)
    sc = _make_sc_kernel()
    word_raw, states_t = sc(wids_t, word_table, trans_flat)
    out_sdb = _tc_assemble(word_raw, states_t, fsa_table, pe)
    return out_sdb.transpose(2, 0, 1)
